# baseline (speedup 1.0000x reference)
import functools

import jax
import jax.numpy as jnp
from jax import lax
from jax.experimental import pallas as pl
from jax.experimental.pallas import tpu as pltpu
from jax.experimental.pallas import tpu_sc as plsc

_V = 8192
_D = 8192
_B = 8192
_NC = 2
_NS = 16
_NW = _NC * _NS
_BW = _B // _NW
_R = 4
_NCHUNK = _BW // _R


def _gather_body(table_hbm, idx_hbm, pos_hbm, out_hbm, idx_v, pos_v,
                 buf_a, buf_b, ga, gb, oa, ob):
    wid = lax.axis_index("s") * _NC + lax.axis_index("c")
    pltpu.sync_copy(idx_hbm.at[wid], idx_v)
    pltpu.sync_copy(pos_hbm.at[wid], pos_v)

    def gather(chunk, buf, sem):
        pltpu.async_copy(table_hbm.at[idx_v.at[chunk]], buf, sem)

    def put(chunk, buf, sem):
        pltpu.async_copy(buf, out_hbm.at[pos_v.at[chunk]], sem)

    def wait_gather(buf, sem):
        pltpu.make_async_copy(table_hbm.at[pl.ds(0, _R)], buf, sem).wait()

    def wait_put(buf, sem):
        pltpu.make_async_copy(buf, out_hbm.at[pl.ds(0, _R)], sem).wait()

    gather(0, buf_a, ga)
    wait_gather(buf_a, ga)
    put(0, buf_a, oa)
    gather(1, buf_b, gb)

    @pl.loop(1, _NCHUNK - 2, step=2)
    def _body(i):
        wait_gather(buf_b, gb)
        put(i, buf_b, ob)
        wait_put(buf_a, oa)
        gather(i + 1, buf_a, ga)
        wait_gather(buf_a, ga)
        put(i + 1, buf_a, oa)
        wait_put(buf_b, ob)
        gather(i + 2, buf_b, gb)

    wait_gather(buf_b, gb)
    put(_NCHUNK - 1, buf_b, ob)
    wait_put(buf_a, oa)
    wait_put(buf_b, ob)


@jax.jit
def _gather(table, idx, pos):
    run = functools.partial(
        pl.kernel,
        mesh=plsc.VectorSubcoreMesh(core_axis_name="c", subcore_axis_name="s"),
        out_type=jax.ShapeDtypeStruct((_B, _D), jnp.float32),
        scratch_types=[
            pltpu.VMEM((_NCHUNK, _R), jnp.int32),
            pltpu.VMEM((_NCHUNK, _R), jnp.int32),
            pltpu.VMEM((_R, _D), jnp.float32),
            pltpu.VMEM((_R, _D), jnp.float32),
            pltpu.SemaphoreType.DMA,
            pltpu.SemaphoreType.DMA,
            pltpu.SemaphoreType.DMA,
            pltpu.SemaphoreType.DMA,
        ],
    )(_gather_body)
    return run(table, idx, pos)


def kernel(x, table):
    flat = x.reshape(_B)
    pos = lax.iota(jnp.int32, _B)
    idx_sorted, pos_sorted = lax.sort([flat, pos], num_keys=1)
    out = _gather(table,
                  idx_sorted.reshape(_NW, _NCHUNK, _R),
                  pos_sorted.reshape(_NW, _NCHUNK, _R))
    return out.reshape(x.shape + (table.shape[1],))


# speedup vs baseline: 1.1203x; 1.1203x over previous
import functools

import jax
import jax.numpy as jnp
from jax import lax
from jax.experimental import pallas as pl
from jax.experimental.pallas import tpu as pltpu
from jax.experimental.pallas import tpu_sc as plsc

_V = 8192
_D = 8192
_B = 8192
_NC = 2
_NS = 16
_NW = _NC * _NS
_BW = _B // _NW
_R = 4
_NCHUNK = _BW // _R


def _gather_body(table_hbm, idx_hbm, out_hbm, idx_v,
                 b0, b1, b2, g0, g1, g2, p0, p1, p2):
    wid = lax.axis_index("s") * _NC + lax.axis_index("c")
    base = wid * _BW
    pltpu.sync_copy(idx_hbm.at[wid], idx_v)

    def gather(chunk, buf, sem):
        pltpu.async_copy(table_hbm.at[idx_v.at[chunk]], buf, sem)

    def put(chunk, buf, sem):
        off = pl.multiple_of(chunk * _R, _R)
        pltpu.async_copy(buf, out_hbm.at[pl.ds(base + off, _R)], sem)

    def wait_gather(buf, sem):
        pltpu.make_async_copy(table_hbm.at[pl.ds(0, _R)], buf, sem).wait()

    def wait_put(buf, sem):
        pltpu.make_async_copy(buf, out_hbm.at[pl.ds(base, _R)], sem).wait()

    gather(0, b2, g2)
    wait_gather(b2, g2)
    put(0, b2, p2)
    gather(1, b0, g0)
    gather(2, b1, g1)
    wait_put(b2, p2)
    gather(3, b2, g2)

    @pl.loop(1, _NCHUNK - 5, step=3)
    def _body(i):
        wait_gather(b0, g0)
        put(i, b0, p0)
        wait_gather(b1, g1)
        put(i + 1, b1, p1)
        wait_gather(b2, g2)
        put(i + 2, b2, p2)
        wait_put(b0, p0)
        gather(i + 3, b0, g0)
        wait_put(b1, p1)
        gather(i + 4, b1, g1)
        wait_put(b2, p2)
        gather(i + 5, b2, g2)

    wait_gather(b0, g0)
    put(_NCHUNK - 3, b0, p0)
    wait_gather(b1, g1)
    put(_NCHUNK - 2, b1, p1)
    wait_gather(b2, g2)
    put(_NCHUNK - 1, b2, p2)
    wait_put(b0, p0)
    wait_put(b1, p1)
    wait_put(b2, p2)


@jax.jit
def _gather(table, idx):
    run = functools.partial(
        pl.kernel,
        mesh=plsc.VectorSubcoreMesh(core_axis_name="c", subcore_axis_name="s"),
        out_type=jax.ShapeDtypeStruct((_B, _D), jnp.float32),
        scratch_types=[
            pltpu.VMEM((_NCHUNK, _R), jnp.int32),
            pltpu.VMEM((_R, _D), jnp.float32),
            pltpu.VMEM((_R, _D), jnp.float32),
            pltpu.VMEM((_R, _D), jnp.float32),
            pltpu.SemaphoreType.DMA,
            pltpu.SemaphoreType.DMA,
            pltpu.SemaphoreType.DMA,
            pltpu.SemaphoreType.DMA,
            pltpu.SemaphoreType.DMA,
            pltpu.SemaphoreType.DMA,
        ],
    )(_gather_body)
    return run(table, idx)


def kernel(x, table):
    idx = x.reshape(_NW, _NCHUNK, _R)
    out = _gather(table, idx)
    return out.reshape(x.shape + (table.shape[1],))


# speedup vs baseline: 1.1236x; 1.0029x over previous
import functools

import jax
import jax.numpy as jnp
from jax import lax
from jax.experimental import pallas as pl
from jax.experimental.pallas import tpu as pltpu
from jax.experimental.pallas import tpu_sc as plsc

_V = 8192
_D = 8192
_B = 8192
_NC = 2
_NS = 16
_NW = _NC * _NS
_BW = _B // _NW
_R = 4
_NCHUNK = _BW // _R


def _gather_body(table_hbm, idx_hbm, out_hbm, idx_v,
                 b0, b1, b2, g0, g1, g2, p0, p1, p2):
    wid = lax.axis_index("s") * _NC + lax.axis_index("c")
    base = wid * _BW
    pltpu.sync_copy(idx_hbm.at[wid], idx_v)

    def gather(chunk, buf, sem):
        pltpu.async_copy(table_hbm.at[idx_v.at[chunk]], buf, sem)

    def put(chunk, buf, sem):
        off = pl.multiple_of(chunk * _R, _R)
        pltpu.async_copy(buf, out_hbm.at[pl.ds(base + off, _R)], sem)

    def wait_gather(buf, sem):
        pltpu.make_async_copy(table_hbm.at[pl.ds(0, _R)], buf, sem).wait()

    def wait_put(buf, sem):
        pltpu.make_async_copy(buf, out_hbm.at[pl.ds(base, _R)], sem).wait()

    gather(0, b2, g2)
    wait_gather(b2, g2)
    put(0, b2, p2)
    gather(1, b0, g0)
    gather(2, b1, g1)
    wait_put(b2, p2)
    gather(3, b2, g2)

    @pl.loop(1, _NCHUNK - 5, step=3)
    def _body(i):
        wait_gather(b0, g0)
        put(i, b0, p0)
        wait_gather(b1, g1)
        put(i + 1, b1, p1)
        wait_put(b0, p0)
        gather(i + 3, b0, g0)
        wait_gather(b2, g2)
        put(i + 2, b2, p2)
        wait_put(b1, p1)
        gather(i + 4, b1, g1)
        wait_put(b2, p2)
        gather(i + 5, b2, g2)

    wait_gather(b0, g0)
    put(_NCHUNK - 3, b0, p0)
    wait_gather(b1, g1)
    put(_NCHUNK - 2, b1, p1)
    wait_gather(b2, g2)
    put(_NCHUNK - 1, b2, p2)
    wait_put(b0, p0)
    wait_put(b1, p1)
    wait_put(b2, p2)


@jax.jit
def _gather(table, idx):
    run = functools.partial(
        pl.kernel,
        mesh=plsc.VectorSubcoreMesh(core_axis_name="c", subcore_axis_name="s"),
        out_type=jax.ShapeDtypeStruct((_B, _D), jnp.float32),
        scratch_types=[
            pltpu.VMEM((_NCHUNK, _R), jnp.int32),
            pltpu.VMEM((_R, _D), jnp.float32),
            pltpu.VMEM((_R, _D), jnp.float32),
            pltpu.VMEM((_R, _D), jnp.float32),
            pltpu.SemaphoreType.DMA,
            pltpu.SemaphoreType.DMA,
            pltpu.SemaphoreType.DMA,
            pltpu.SemaphoreType.DMA,
            pltpu.SemaphoreType.DMA,
            pltpu.SemaphoreType.DMA,
        ],
    )(_gather_body)
    return run(table, idx)


def kernel(x, table):
    idx = x.reshape(_NW, _NCHUNK, _R)
    out = _gather(table, idx)
    return out.reshape(x.shape + (table.shape[1],))
